# baseline (speedup 1.0000x reference)
import functools

import jax
import jax.numpy as jnp
from jax import lax
from jax.experimental import pallas as pl
from jax.experimental.pallas import tpu as pltpu
from jax.experimental.pallas import tpu_sc as plsc

N = 10000
D = 128
NG = 256
NP = 10240
N_ACC = 10016
TRASH = N_ACC - 1
ROWS_ACC_SUB = N_ACC // 16
E = 320000
EBLK = 2560
E_PAD = EBLK * 128
NC = 2
NS = 16
ROWS_PER_SUB = NP // NS
EPW = E_PAD // 32
RBLK = EPW // 128
RING = 6
GDEP = 4
SDEP = 2
BN = 512
GRID = NP // BN

@functools.cache
def _mesh():
    return plsc.VectorSubcoreMesh(core_axis_name="c", subcore_axis_name="s",
                                  num_cores=NC, num_subcores=NS)
_DOT = dict(preferred_element_type=jnp.float32,
            precision=jax.lax.Precision.HIGHEST)


@functools.cache
def _sc_msg_kernel():
    return pl.kernel(
        _sc_msg_body,
        out_type=jax.ShapeDtypeStruct((NC, 3, N_ACC, 64), jnp.float32),
        mesh=_mesh(),
        compiler_params=pltpu.CompilerParams(use_tc_tiling_on_sc=False),
        scratch_types=[
            pltpu.VMEM((RBLK, 128), jnp.int32),
            pltpu.VMEM((RBLK, 128), jnp.int32),
            pltpu.VMEM((16,), jnp.int32),
            pltpu.VMEM((RING, 128, 64), jnp.float32),
            pltpu.VMEM_SHARED((N_ACC, 64), jnp.float32),
            pltpu.SemaphoreType.DMA,
            pltpu.SemaphoreType.DMA,
        ],
    )


def _sc_msg_body(tbl_hbm, src_hbm, dst_hbm, cnt_hbm, zeros_hbm, out_hbm,
                 sidx, didx, cbuf, rows, acc, gsem, ssem):
    c = lax.axis_index("c")
    s = lax.axis_index("s")

    for t in range(3):
        pltpu.sync_copy(zeros_hbm.at[pl.ds(s * ROWS_ACC_SUB, ROWS_ACC_SUB)],
                        acc.at[pl.ds(s * ROWS_ACC_SUB, ROWS_ACC_SUB)])
        plsc.subcore_barrier()
        for rr in range(2):
            r = s * 2 + rr
            pltpu.sync_copy(cnt_hbm.at[t * 32 + r], cbuf)
            nb = cbuf[...][0]
            pltpu.sync_copy(src_hbm.at[t, pl.ds(r * RBLK, RBLK)], sidx)
            pltpu.sync_copy(dst_hbm.at[t, pl.ds(r * RBLK, RBLK)], didx)

            def step(j, carry):
                p = lax.rem(j, RING)
                pm = lax.rem(j + (RING - GDEP), RING)

                @pl.when(j >= GDEP + SDEP)
                def _():
                    pltpu.make_async_copy(zeros_hbm.at[pl.ds(0, 128)],
                                          rows.at[0], ssem).wait()

                @pl.when(j < nb)
                def _():
                    pltpu.async_copy(tbl_hbm.at[c, t].at[sidx.at[j]], rows.at[p], gsem)

                @pl.when(jnp.logical_and(j >= GDEP, j < nb + GDEP))
                def _():
                    pltpu.make_async_copy(zeros_hbm.at[pl.ds(0, 128)],
                                          rows.at[0], gsem).wait()
                    pltpu.async_copy(rows.at[pm], acc.at[didx.at[j - GDEP]],
                                     ssem, add=True)
                return carry
            lax.fori_loop(0, nb + GDEP + SDEP, step, 0)
        plsc.subcore_barrier()
        pltpu.sync_copy(acc.at[pl.ds(s * ROWS_ACC_SUB, ROWS_ACC_SUB)],
                        out_hbm.at[c, t, pl.ds(s * ROWS_ACC_SUB, ROWS_ACC_SUB)])


@functools.cache
def _sc_prep_kernel():
    return pl.kernel(
        _sc_prep_body,
        out_type=(jax.ShapeDtypeStruct((3, 32, EPW), jnp.int32),
                  jax.ShapeDtypeStruct((3, 32, EPW), jnp.int32),
                  jax.ShapeDtypeStruct((96, 16), jnp.int32),
                  jax.ShapeDtypeStruct((NC, NP, 16), jnp.float32)),
        mesh=_mesh(),
        compiler_params=pltpu.CompilerParams(use_tc_tiling_on_sc=False,
                                             needs_layout_passes=False),
        scratch_types=[
            pltpu.VMEM((EPW,), jnp.int32),
            pltpu.VMEM((3, EPW), jnp.int32),
            pltpu.VMEM((3, EPW), jnp.int32),
            pltpu.VMEM((3, EPW), jnp.int32),
            pltpu.VMEM((16,), jnp.int32),
            pltpu.VMEM((2, 128), jnp.int32),
            pltpu.VMEM((2, 128, 16), jnp.float32),
            pltpu.VMEM((ROWS_PER_SUB, 16), jnp.float32),
            pltpu.VMEM_SHARED((NP, 16), jnp.float32),
        ],
    )


def _sc_prep_body(src_hbm, dst_hbm, dstr_hbm, w_hbm,
                  osrc_hbm, odst_hbm, cnt_hbm, deg_hbm,
                  sflat, dflat, osrc, odst, cbuf, dstv, wv, zbuf, dacc):
    c = lax.axis_index("c")
    s = lax.axis_index("s")
    w = s * NC + c

    pltpu.sync_copy(src_hbm.at[pl.ds(w * EPW, EPW)], sflat)
    for t in range(3):
        pltpu.sync_copy(dst_hbm.at[t, pl.ds(w * EPW, EPW)], dflat.at[t])

    def prefill(k, carry):
        z16 = jnp.zeros((16,), jnp.int32)
        t16 = jnp.full((16,), TRASH, jnp.int32)
        for t in range(3):
            osrc[t, pl.ds(k * 16, 16)] = z16
            odst[t, pl.ds(k * 16, 16)] = t16
        return carry
    lax.fori_loop(0, EPW // 16, prefill, 0)

    zero16 = jnp.zeros((16,), jnp.int32)

    def chunk(k, cnts):
        sv = sflat[pl.ds(k * 16, 16)]
        new = []
        for t in range(3):
            dv = dflat[t, pl.ds(k * 16, 16)]
            m = dv != TRASH
            mi = jnp.where(m, 1, 0).astype(jnp.int32)
            pref = plsc.cumsum(mi)
            pos = cnts[t] + pref - mi
            plsc.store_scatter(osrc.at[t], [pos], sv, mask=m)
            plsc.store_scatter(odst.at[t], [pos], dv, mask=m)
            pc = plsc.all_reduce_population_count(m)
            if pc.ndim == 0:
                pc = jax.lax.broadcast_in_dim(pc, (16,), ())
            new.append(cnts[t] + pc)
        return tuple(new)
    cnts = lax.fori_loop(0, EPW // 16, chunk, (zero16, zero16, zero16))

    for t in range(3):
        pltpu.sync_copy(osrc.at[t], osrc_hbm.at[t, w])
        pltpu.sync_copy(odst.at[t], odst_hbm.at[t, w])
        cbuf[...] = (cnts[t] + 127) // 128
        pltpu.sync_copy(cbuf, cnt_hbm.at[t * 32 + w])

    def zrow(i, carry):
        zbuf[i] = jnp.zeros((16,), jnp.float32)
        return carry
    lax.fori_loop(0, ROWS_PER_SUB, zrow, 0)
    pltpu.sync_copy(zbuf, dacc.at[pl.ds(s * ROWS_PER_SUB, ROWS_PER_SUB)])
    plsc.subcore_barrier()

    nblk = EBLK // (NC * NS)

    def grp(g, carry):
        b0 = w * nblk + g * 2
        pltpu.sync_copy(dstr_hbm.at[pl.ds(b0, 2)], dstv)
        pltpu.sync_copy(w_hbm.at[pl.ds(b0, 2)], wv)
        for j in range(2):
            pltpu.sync_copy(wv.at[j], dacc.at[dstv.at[j]], add=True)
        return carry
    lax.fori_loop(0, nblk // 2, grp, 0)
    plsc.subcore_barrier()
    pltpu.sync_copy(dacc.at[pl.ds(s * ROWS_PER_SUB, ROWS_PER_SUB)],
                    deg_hbm.at[c, pl.ds(s * ROWS_PER_SUB, ROWS_PER_SUB)])


def _tc_prep(x_pad, degparts):
    def body(x_ref, dg_ref, tbl_ref, dis_ref):
        deg = dg_ref[0] + dg_ref[1]
        dis = jnp.where(deg > 0, 1.0 / jnp.sqrt(jnp.where(deg > 0, deg, 1.0)), 0.0)
        dis_ref[...] = dis[:, :8]
        for cc in range(2):
            for t in range(3):
                tbl_ref[cc, t] = dis[:, t:t + 1] * x_ref[:, cc * 64:(cc + 1) * 64]

    return pl.pallas_call(
        body,
        grid=(GRID,),
        in_specs=[pl.BlockSpec((BN, 128), lambda i: (i, 0)),
                  pl.BlockSpec((2, BN, 16), lambda i: (0, i, 0))],
        out_specs=[pl.BlockSpec((2, 3, BN, 64), lambda i: (0, 0, i, 0)),
                   pl.BlockSpec((BN, 8), lambda i: (i, 0))],
        out_shape=[jax.ShapeDtypeStruct((2, 3, NP, 64), jnp.float32),
                   jax.ShapeDtypeStruct((NP, 8), jnp.float32)],
    )(x_pad, degparts)


def _tc_combine(mparts, dis8, h, wst, bvec):
    def body(mp_ref, dis_ref, h_ref, w_ref, b_ref, a_ref, st_ref):
        i = pl.program_id(0)
        dis = dis_ref[...]
        acc = jax.nn.relu(
            jax.lax.dot_general(h_ref[...], w_ref[3], (((1,), (1,)), ((), ())), **_DOT)
            + b_ref[3])
        for t in range(3):
            m = jnp.concatenate([mp_ref[0, t], mp_ref[1, t]], axis=1)
            pre = dis[:, t:t + 1] * m
            acc = acc + jax.nn.relu(
                jax.lax.dot_general(pre, w_ref[t], (((1,), (1,)), ((), ())), **_DOT)
                + b_ref[t])
        rowid = i * BN + jax.lax.broadcasted_iota(jnp.int32, (BN, 1), 0)
        acc = jnp.where(rowid < N, acc, 0.0)
        a_ref[...] = acc
        st = jnp.concatenate([jnp.sum(acc.reshape(BN // 8, 8, 128), axis=0),
                              jnp.sum((acc * acc).reshape(BN // 8, 8, 128), axis=0)],
                             axis=0)

        @pl.when(i == 0)
        def _():
            st_ref[...] = st

        @pl.when(i > 0)
        def _():
            st_ref[...] += st

    return pl.pallas_call(
        body,
        grid=(GRID,),
        in_specs=[pl.BlockSpec((2, 3, BN, 64), lambda i: (0, 0, i, 0)),
                  pl.BlockSpec((BN, 8), lambda i: (i, 0)),
                  pl.BlockSpec((BN, 128), lambda i: (i, 0)),
                  pl.BlockSpec((4, 128, 128), lambda i: (0, 0, 0)),
                  pl.BlockSpec((8, 128), lambda i: (0, 0))],
        out_specs=[pl.BlockSpec((BN, 128), lambda i: (i, 0)),
                   pl.BlockSpec((16, 128), lambda i: (0, 0))],
        out_shape=[jax.ShapeDtypeStruct((NP, 128), jnp.float32),
                   jax.ShapeDtypeStruct((16, 128), jnp.float32)],
    )(mparts, dis8, h, wst, bvec)


def _tc_norm(a, stats, bvec, dis8):
    def body(a_ref, st_ref, b_ref, dis_ref, h_ref, tbl_ref, vs_ref):
        p = pl.program_id(0)
        i = pl.program_id(1)
        mean = jnp.sum(st_ref[...][:8], axis=0) * (1.0 / N)
        rowid = i * BN + jax.lax.broadcasted_iota(jnp.int32, (BN, 1), 0)
        d = jnp.where(rowid < N, a_ref[...] - mean, 0.0)

        @pl.when(p == 0)
        def _():
            sq = jnp.sum((d * d).reshape(BN // 8, 8, 128), axis=0)

            @pl.when(i == 0)
            def _():
                vs_ref[...] = sq

            @pl.when(i > 0)
            def _():
                vs_ref[...] += sq

        @pl.when(p == 1)
        def _():
            var = jnp.sum(vs_ref[...], axis=0) * (1.0 / N)
            rstd = 1.0 / jnp.sqrt(var + 1e-5)
            hh = jax.nn.relu((a_ref[...] - mean) * rstd * b_ref[4] + b_ref[5])
            h_ref[...] = hh
            dis = dis_ref[...]
            for cc in range(2):
                for t in range(3):
                    tbl_ref[cc, t] = dis[:, t:t + 1] * hh[:, cc * 64:(cc + 1) * 64]

    return pl.pallas_call(
        body,
        grid=(2, GRID),
        in_specs=[pl.BlockSpec((BN, 128), lambda p, i: (i, 0)),
                  pl.BlockSpec((16, 128), lambda p, i: (0, 0)),
                  pl.BlockSpec((8, 128), lambda p, i: (0, 0)),
                  pl.BlockSpec((BN, 8), lambda p, i: (i, 0))],
        out_specs=[pl.BlockSpec((BN, 128), lambda p, i: (i, 0)),
                   pl.BlockSpec((2, 3, BN, 64), lambda p, i: (0, 0, i, 0)),
                   pl.BlockSpec((8, 128), lambda p, i: (0, 0))],
        out_shape=[jax.ShapeDtypeStruct((NP, 128), jnp.float32),
                   jax.ShapeDtypeStruct((2, 3, NP, 64), jnp.float32),
                   jax.ShapeDtypeStruct((8, 128), jnp.float32)],
    )(a, stats, bvec, dis8)[:2]


def _tc_final(a2, stats2, bvec, batch_r, w1, w2p, fcb):
    def body(a_ref, st_ref, b_ref, bt_ref, w1_ref, w2_ref, fb_ref,
             g_ref, o_ref, vs_ref):
        p = pl.program_id(0)
        i = pl.program_id(1)
        mean = jnp.sum(st_ref[...][:8], axis=0) * (1.0 / N)
        rowid = i * BN + jax.lax.broadcasted_iota(jnp.int32, (BN, 1), 0)
        d = jnp.where(rowid < N, a_ref[...] - mean, 0.0)

        @pl.when(p == 0)
        def _():
            sq = jnp.sum((d * d).reshape(BN // 8, 8, 128), axis=0)

            @pl.when(i == 0)
            def _():
                vs_ref[...] = sq

            @pl.when(i > 0)
            def _():
                vs_ref[...] += sq

        @pl.when(p == 1)
        def _():
            var = jnp.sum(vs_ref[...], axis=0) * (1.0 / N)
            rstd = 1.0 / jnp.sqrt(var + 1e-5)
            h2 = jax.nn.relu((a_ref[...] - mean) * rstd * b_ref[4] + b_ref[5])
            bb = bt_ref[0, 0]
            oh = (jax.lax.broadcasted_iota(jnp.int32, (NG, BN), 0)
                  == bb[None, :]).astype(jnp.float32)
            part = jax.lax.dot_general(oh, h2, (((1,), (0,)), ((), ())), **_DOT)

            @pl.when(i == 0)
            def _():
                g_ref[...] = part

            @pl.when(i > 0)
            def _():
                g_ref[...] += part

            @pl.when(i == GRID - 1)
            def _():
                gg = g_ref[...]
                z = jax.nn.relu(
                    jax.lax.dot_general(gg, w1_ref[...], (((1,), (1,)), ((), ())),
                                        **_DOT) + fb_ref[0])
                o_ref[...] = jax.lax.dot_general(
                    z, w2_ref[...], (((1,), (1,)), ((), ())), **_DOT) + fb_ref[1]

    return pl.pallas_call(
        body,
        grid=(2, GRID),
        in_specs=[pl.BlockSpec((BN, 128), lambda p, i: (i, 0)),
                  pl.BlockSpec((16, 128), lambda p, i: (0, 0)),
                  pl.BlockSpec((8, 128), lambda p, i: (0, 0)),
                  pl.BlockSpec((1, 1, BN), lambda p, i: (i, 0, 0)),
                  pl.BlockSpec((128, 128), lambda p, i: (0, 0)),
                  pl.BlockSpec((128, 128), lambda p, i: (0, 0)),
                  pl.BlockSpec((8, 128), lambda p, i: (0, 0))],
        out_specs=[pl.BlockSpec((NG, 128), lambda p, i: (0, 0)),
                   pl.BlockSpec((NG, 128), lambda p, i: (0, 0)),
                   pl.BlockSpec((8, 128), lambda p, i: (0, 0))],
        out_shape=[jax.ShapeDtypeStruct((NG, 128), jnp.float32),
                   jax.ShapeDtypeStruct((NG, 128), jnp.float32),
                   jax.ShapeDtypeStruct((8, 128), jnp.float32)],
    )(a2, stats2, bvec, batch_r, w1, w2p, fcb)


def kernel(x, edge_index, edge_attr, batch, mask, params):
    f32, i32 = jnp.float32, jnp.int32
    src = edge_index[0].astype(i32)
    dst = edge_index[1].astype(i32)
    epad = E_PAD - E

    w3 = edge_attr[:, :3] > 0.5
    dst3 = jnp.where(w3.T, dst[None, :], TRASH).astype(i32)
    src_f = jnp.concatenate([src, jnp.zeros((epad,), i32)])
    dst_r = jnp.concatenate([dst, jnp.full((epad,), TRASH, i32)]).reshape(EBLK, 128)
    dst3_f = jnp.concatenate([dst3, jnp.full((3, epad), TRASH, i32)], axis=1)
    wrow_r = jnp.concatenate(
        [jnp.pad(edge_attr.astype(f32), ((0, 0), (0, 12))),
         jnp.zeros((epad, 16), f32)], axis=0).reshape(EBLK, 128, 16)

    x_pad = jnp.pad(x.astype(f32), ((0, NP - N), (0, 0)))
    batch_r = jnp.concatenate([batch.astype(i32),
                               jnp.full((NP - N,), NG, i32)]).reshape(GRID, 1, BN)

    p = params
    wst = [jnp.stack([p[f"W_s{i}"], p[f"W_d{i}"], p[f"W_t{i}"], p[f"W_id{i}"]])
           for i in range(2)]
    bvec = [jnp.stack([p[f"b_s{i}"], p[f"b_d{i}"], p[f"b_t{i}"], p[f"b_id{i}"],
                       p[f"bn_g{i}"], p[f"bn_b{i}"],
                       jnp.zeros((128,), f32), jnp.zeros((128,), f32)])
            for i in range(2)]
    w2p = jnp.zeros((128, 128), f32).at[:2].set(p["W_fc2"])
    fcb = jnp.zeros((8, 128), f32)
    fcb = fcb.at[0].set(p["b_fc1"])
    fcb = fcb.at[1, :2].set(p["b_fc2"])

    z64 = jnp.zeros((N_ACC, 64), f32)
    csrc_f, cdst_f, cnts, degparts = _sc_prep_kernel()(src_f, dst3_f, dst_r, wrow_r)
    csrc = csrc_f.reshape(3, EBLK, 128)
    cdst = cdst_f.reshape(3, EBLK, 128)
    tbl1, dis8 = _tc_prep(x_pad, degparts)
    mp1 = _sc_msg_kernel()(tbl1, csrc, cdst, cnts, z64)
    a1, st1 = _tc_combine(mp1, dis8, x_pad, wst[0], bvec[0])
    h1, tbl2 = _tc_norm(a1, st1, bvec[0], dis8)
    mp2 = _sc_msg_kernel()(tbl2, csrc, cdst, cnts, z64)
    a2, st2 = _tc_combine(mp2, dis8, h1, wst[1], bvec[1])
    _, res, _ = _tc_final(a2, st2, bvec[1], batch_r, p["W_fc1"], w2p, fcb)
    return res[:, :2]


# speedup vs baseline: 1.1017x; 1.1017x over previous
import functools

import jax
import jax.numpy as jnp
from jax import lax
from jax.experimental import pallas as pl
from jax.experimental.pallas import tpu as pltpu
from jax.experimental.pallas import tpu_sc as plsc

N = 10000
D = 128
NG = 256
NP = 10240
N_ACC = 10016
TRASH = N_ACC - 1
ROWS_ACC_SUB = N_ACC // 16
E = 320000
EBLK = 2560
E_PAD = EBLK * 128
NC = 2
NS = 16
BLK_PER_SUB = EBLK // (NC * NS) * NC
ROWS_PER_SUB = NP // NS
BPS = EBLK // NS
EPW = E_PAD // 32
RBLK = EPW // 128
RING = 6
GDEP = 4
SDEP = 2
BN = 512
GRID = NP // BN

@functools.cache
def _mesh():
    return plsc.VectorSubcoreMesh(core_axis_name="c", subcore_axis_name="s",
                                  num_cores=NC, num_subcores=NS)
_DOT = dict(preferred_element_type=jnp.float32)


@functools.cache
def _sc_deg_kernel():
    return pl.kernel(
        _sc_deg_body,
        out_type=jax.ShapeDtypeStruct((NC, NP, 16), jnp.float32),
        mesh=_mesh(),
        compiler_params=pltpu.CompilerParams(use_tc_tiling_on_sc=False),
        scratch_types=[
            pltpu.VMEM((4, 128), jnp.int32),
            pltpu.VMEM((4, 128, 16), jnp.float32),
            pltpu.VMEM((ROWS_PER_SUB, 16), jnp.float32),
            pltpu.VMEM_SHARED((NP, 16), jnp.float32),
        ],
    )


def _sc_deg_body(dst_hbm, w_hbm, out_hbm, dstv, wv, zbuf, acc):
    c = lax.axis_index("c")
    s = lax.axis_index("s")
    wid = s * NC + c

    def zrow(i, carry):
        zbuf[i] = jnp.zeros((16,), jnp.float32)
        return carry
    lax.fori_loop(0, ROWS_PER_SUB, zrow, 0)
    pltpu.sync_copy(zbuf, acc.at[pl.ds(s * ROWS_PER_SUB, ROWS_PER_SUB)])
    plsc.subcore_barrier()

    nblk = EBLK // (NC * NS)

    def grp(g, carry):
        b0 = wid * nblk + g * 4
        pltpu.sync_copy(dst_hbm.at[pl.ds(b0, 4)], dstv)
        pltpu.sync_copy(w_hbm.at[pl.ds(b0, 4)], wv)
        for j in range(4):
            pltpu.sync_copy(wv.at[j], acc.at[dstv.at[j]], add=True)
        return carry
    lax.fori_loop(0, nblk // 4, grp, 0)
    plsc.subcore_barrier()
    pltpu.sync_copy(acc.at[pl.ds(s * ROWS_PER_SUB, ROWS_PER_SUB)],
                    out_hbm.at[c, pl.ds(s * ROWS_PER_SUB, ROWS_PER_SUB)])


@functools.cache
def _sc_msg_kernel():
    return pl.kernel(
        _sc_msg_body,
        out_type=jax.ShapeDtypeStruct((NC, 3, N_ACC, 64), jnp.float32),
        mesh=_mesh(),
        compiler_params=pltpu.CompilerParams(use_tc_tiling_on_sc=False),
        scratch_types=[
            pltpu.VMEM((RBLK, 128), jnp.int32),
            pltpu.VMEM((RBLK, 128), jnp.int32),
            pltpu.VMEM((16,), jnp.int32),
            pltpu.VMEM((RING, 128, 64), jnp.float32),
            pltpu.VMEM_SHARED((N_ACC, 64), jnp.float32),
            pltpu.SemaphoreType.DMA,
            pltpu.SemaphoreType.DMA,
        ],
    )


def _sc_msg_body(tbl_hbm, src_hbm, dst_hbm, cnt_hbm, zeros_hbm, out_hbm,
                 sidx, didx, cbuf, rows, acc, gsem, ssem):
    c = lax.axis_index("c")
    s = lax.axis_index("s")

    for t in range(3):
        pltpu.sync_copy(zeros_hbm.at[pl.ds(s * ROWS_ACC_SUB, ROWS_ACC_SUB)],
                        acc.at[pl.ds(s * ROWS_ACC_SUB, ROWS_ACC_SUB)])
        plsc.subcore_barrier()
        for rr in range(2):
            r = s * 2 + rr
            pltpu.sync_copy(cnt_hbm.at[t * 32 + r], cbuf)
            nb = cbuf[...][0]
            pltpu.sync_copy(src_hbm.at[t, pl.ds(r * RBLK, RBLK)], sidx)
            pltpu.sync_copy(dst_hbm.at[t, pl.ds(r * RBLK, RBLK)], didx)

            def step(j, carry):
                p = lax.rem(j, RING)
                pm = lax.rem(j + (RING - GDEP), RING)

                @pl.when(j >= GDEP + SDEP)
                def _():
                    pltpu.make_async_copy(zeros_hbm.at[pl.ds(0, 128)],
                                          rows.at[0], ssem).wait()

                @pl.when(j < nb)
                def _():
                    pltpu.async_copy(tbl_hbm.at[c, t].at[sidx.at[j]], rows.at[p], gsem)

                @pl.when(jnp.logical_and(j >= GDEP, j < nb + GDEP))
                def _():
                    pltpu.make_async_copy(zeros_hbm.at[pl.ds(0, 128)],
                                          rows.at[0], gsem).wait()
                    pltpu.async_copy(rows.at[pm], acc.at[didx.at[j - GDEP]],
                                     ssem, add=True)
                return carry
            lax.fori_loop(0, nb + GDEP + SDEP, step, 0)
        plsc.subcore_barrier()
        pltpu.sync_copy(acc.at[pl.ds(s * ROWS_ACC_SUB, ROWS_ACC_SUB)],
                        out_hbm.at[c, t, pl.ds(s * ROWS_ACC_SUB, ROWS_ACC_SUB)])


@functools.cache
def _sc_compact_kernel():
    return pl.kernel(
        _sc_compact_body,
        out_type=(jax.ShapeDtypeStruct((3, 32, EPW), jnp.int32),
                  jax.ShapeDtypeStruct((3, 32, EPW), jnp.int32),
                  jax.ShapeDtypeStruct((96, 16), jnp.int32)),
        mesh=_mesh(),
        compiler_params=pltpu.CompilerParams(use_tc_tiling_on_sc=False,
                                             needs_layout_passes=False),
        scratch_types=[
            pltpu.VMEM((EPW,), jnp.int32),
            pltpu.VMEM((3, EPW), jnp.int32),
            pltpu.VMEM((3, EPW), jnp.int32),
            pltpu.VMEM((3, EPW), jnp.int32),
            pltpu.VMEM((16,), jnp.int32),
        ],
    )


def _sc_compact_body(src_hbm, dst_hbm, osrc_hbm, odst_hbm, cnt_hbm,
                     sflat, dflat, osrc, odst, cbuf):
    c = lax.axis_index("c")
    s = lax.axis_index("s")
    w = s * NC + c

    pltpu.sync_copy(src_hbm.at[pl.ds(w * EPW, EPW)], sflat)
    for t in range(3):
        pltpu.sync_copy(dst_hbm.at[t, pl.ds(w * EPW, EPW)], dflat.at[t])

    def prefill(k, carry):
        z16 = jnp.zeros((16,), jnp.int32)
        t16 = jnp.full((16,), TRASH, jnp.int32)
        for t in range(3):
            osrc[t, pl.ds(k * 16, 16)] = z16
            odst[t, pl.ds(k * 16, 16)] = t16
        return carry
    lax.fori_loop(0, EPW // 16, prefill, 0)

    zero16 = jnp.zeros((16,), jnp.int32)

    def chunk(k, cnts):
        sv = sflat[pl.ds(k * 16, 16)]
        new = []
        for t in range(3):
            dv = dflat[t, pl.ds(k * 16, 16)]
            m = dv != TRASH
            mi = jnp.where(m, 1, 0).astype(jnp.int32)
            pref = plsc.cumsum(mi)
            pos = cnts[t] + pref - mi
            plsc.store_scatter(osrc.at[t], [pos], sv, mask=m)
            plsc.store_scatter(odst.at[t], [pos], dv, mask=m)
            pc = plsc.all_reduce_population_count(m)
            if pc.ndim == 0:
                pc = jax.lax.broadcast_in_dim(pc, (16,), ())
            new.append(cnts[t] + pc)
        return tuple(new)
    cnts = lax.fori_loop(0, EPW // 16, chunk, (zero16, zero16, zero16))

    for t in range(3):
        pltpu.sync_copy(osrc.at[t], osrc_hbm.at[t, w])
        pltpu.sync_copy(odst.at[t], odst_hbm.at[t, w])
        cbuf[...] = (cnts[t] + 127) // 128
        pltpu.sync_copy(cbuf, cnt_hbm.at[t * 32 + w])


def _tc_prep(x_pad, degparts):
    def body(x_ref, dg_ref, tbl_ref, dis_ref):
        deg = dg_ref[0] + dg_ref[1]
        dis = jnp.where(deg > 0, 1.0 / jnp.sqrt(jnp.where(deg > 0, deg, 1.0)), 0.0)
        dis_ref[...] = dis[:, :8]
        for cc in range(2):
            for t in range(3):
                tbl_ref[cc, t] = dis[:, t:t + 1] * x_ref[:, cc * 64:(cc + 1) * 64]

    return pl.pallas_call(
        body,
        grid=(GRID,),
        in_specs=[pl.BlockSpec((BN, 128), lambda i: (i, 0)),
                  pl.BlockSpec((2, BN, 16), lambda i: (0, i, 0))],
        out_specs=[pl.BlockSpec((2, 3, BN, 64), lambda i: (0, 0, i, 0)),
                   pl.BlockSpec((BN, 8), lambda i: (i, 0))],
        out_shape=[jax.ShapeDtypeStruct((2, 3, NP, 64), jnp.float32),
                   jax.ShapeDtypeStruct((NP, 8), jnp.float32)],
    )(x_pad, degparts)


def _tc_combine(mparts, dis8, h, wst, bvec):
    def body(mp_ref, dis_ref, h_ref, w_ref, b_ref, a_ref, st_ref):
        i = pl.program_id(0)
        dis = dis_ref[...]
        acc = jax.nn.relu(
            jax.lax.dot_general(h_ref[...], w_ref[3], (((1,), (1,)), ((), ())), **_DOT)
            + b_ref[3])
        for t in range(3):
            m = jnp.concatenate([mp_ref[0, t], mp_ref[1, t]], axis=1)
            pre = dis[:, t:t + 1] * m
            acc = acc + jax.nn.relu(
                jax.lax.dot_general(pre, w_ref[t], (((1,), (1,)), ((), ())), **_DOT)
                + b_ref[t])
        rowid = i * BN + jax.lax.broadcasted_iota(jnp.int32, (BN, 1), 0)
        acc = jnp.where(rowid < N, acc, 0.0)
        a_ref[...] = acc
        st = jnp.concatenate([jnp.sum(acc.reshape(BN // 8, 8, 128), axis=0),
                              jnp.sum((acc * acc).reshape(BN // 8, 8, 128), axis=0)],
                             axis=0)

        @pl.when(i == 0)
        def _():
            st_ref[...] = st

        @pl.when(i > 0)
        def _():
            st_ref[...] += st

    return pl.pallas_call(
        body,
        grid=(GRID,),
        in_specs=[pl.BlockSpec((2, 3, BN, 64), lambda i: (0, 0, i, 0)),
                  pl.BlockSpec((BN, 8), lambda i: (i, 0)),
                  pl.BlockSpec((BN, 128), lambda i: (i, 0)),
                  pl.BlockSpec((4, 128, 128), lambda i: (0, 0, 0)),
                  pl.BlockSpec((8, 128), lambda i: (0, 0))],
        out_specs=[pl.BlockSpec((BN, 128), lambda i: (i, 0)),
                   pl.BlockSpec((16, 128), lambda i: (0, 0))],
        out_shape=[jax.ShapeDtypeStruct((NP, 128), jnp.float32),
                   jax.ShapeDtypeStruct((16, 128), jnp.float32)],
    )(mparts, dis8, h, wst, bvec)


def _tc_norm(a, stats, bvec, dis8):
    def body(a_ref, st_ref, b_ref, dis_ref, h_ref, tbl_ref):
        st = st_ref[...]
        mean = jnp.sum(st[:8], axis=0) * (1.0 / N)
        ex2 = jnp.sum(st[8:], axis=0) * (1.0 / N)
        rstd = 1.0 / jnp.sqrt(ex2 - mean * mean + 1e-5)
        hh = jax.nn.relu((a_ref[...] - mean) * rstd * b_ref[4] + b_ref[5])
        h_ref[...] = hh
        dis = dis_ref[...]
        for cc in range(2):
            for t in range(3):
                tbl_ref[cc, t] = dis[:, t:t + 1] * hh[:, cc * 64:(cc + 1) * 64]

    return pl.pallas_call(
        body,
        grid=(GRID,),
        in_specs=[pl.BlockSpec((BN, 128), lambda i: (i, 0)),
                  pl.BlockSpec((16, 128), lambda i: (0, 0)),
                  pl.BlockSpec((8, 128), lambda i: (0, 0)),
                  pl.BlockSpec((BN, 8), lambda i: (i, 0))],
        out_specs=[pl.BlockSpec((BN, 128), lambda i: (i, 0)),
                   pl.BlockSpec((2, 3, BN, 64), lambda i: (0, 0, i, 0))],
        out_shape=[jax.ShapeDtypeStruct((NP, 128), jnp.float32),
                   jax.ShapeDtypeStruct((2, 3, NP, 64), jnp.float32)],
    )(a, stats, bvec, dis8)


def _tc_final(a2, stats2, bvec, batch_r, w1, w2p, fcb):
    def body(a_ref, st_ref, b_ref, bt_ref, w1_ref, w2_ref, fb_ref, g_ref, o_ref):
        i = pl.program_id(0)
        st = st_ref[...]
        mean = jnp.sum(st[:8], axis=0) * (1.0 / N)
        ex2 = jnp.sum(st[8:], axis=0) * (1.0 / N)
        rstd = 1.0 / jnp.sqrt(ex2 - mean * mean + 1e-5)
        h2 = jax.nn.relu((a_ref[...] - mean) * rstd * b_ref[4] + b_ref[5])
        bb = bt_ref[0, 0]
        oh = (jax.lax.broadcasted_iota(jnp.int32, (NG, BN), 0)
              == bb[None, :]).astype(jnp.float32)
        part = jax.lax.dot_general(oh, h2, (((1,), (0,)), ((), ())), **_DOT)

        @pl.when(i == 0)
        def _():
            g_ref[...] = part

        @pl.when(i > 0)
        def _():
            g_ref[...] += part

        @pl.when(i == GRID - 1)
        def _():
            gg = g_ref[...]
            z = jax.nn.relu(
                jax.lax.dot_general(gg, w1_ref[...], (((1,), (1,)), ((), ())), **_DOT)
                + fb_ref[0])
            o_ref[...] = jax.lax.dot_general(
                z, w2_ref[...], (((1,), (1,)), ((), ())), **_DOT) + fb_ref[1]

    return pl.pallas_call(
        body,
        grid=(GRID,),
        in_specs=[pl.BlockSpec((BN, 128), lambda i: (i, 0)),
                  pl.BlockSpec((16, 128), lambda i: (0, 0)),
                  pl.BlockSpec((8, 128), lambda i: (0, 0)),
                  pl.BlockSpec((1, 1, BN), lambda i: (i, 0, 0)),
                  pl.BlockSpec((128, 128), lambda i: (0, 0)),
                  pl.BlockSpec((128, 128), lambda i: (0, 0)),
                  pl.BlockSpec((8, 128), lambda i: (0, 0))],
        out_specs=[pl.BlockSpec((NG, 128), lambda i: (0, 0)),
                   pl.BlockSpec((NG, 128), lambda i: (0, 0))],
        out_shape=[jax.ShapeDtypeStruct((NG, 128), jnp.float32),
                   jax.ShapeDtypeStruct((NG, 128), jnp.float32)],
    )(a2, stats2, bvec, batch_r, w1, w2p, fcb)


def kernel(x, edge_index, edge_attr, batch, mask, params):
    f32, i32 = jnp.float32, jnp.int32
    src = edge_index[0].astype(i32)
    dst = edge_index[1].astype(i32)
    epad = E_PAD - E

    w3 = edge_attr[:, :3] > 0.5
    dst3 = jnp.where(w3.T, dst[None, :], TRASH).astype(i32)
    src_f = jnp.concatenate([src, jnp.zeros((epad,), i32)])
    dst_r = jnp.concatenate([dst, jnp.full((epad,), TRASH, i32)]).reshape(EBLK, 128)
    dst3_f = jnp.concatenate([dst3, jnp.full((3, epad), TRASH, i32)], axis=1)
    wrow_r = jnp.concatenate(
        [jnp.pad(edge_attr.astype(f32), ((0, 0), (0, 12))),
         jnp.zeros((epad, 16), f32)], axis=0).reshape(EBLK, 128, 16)

    x_pad = jnp.pad(x.astype(f32), ((0, NP - N), (0, 0)))
    batch_r = jnp.concatenate([batch.astype(i32),
                               jnp.full((NP - N,), NG, i32)]).reshape(GRID, 1, BN)

    p = params
    wst = [jnp.stack([p[f"W_s{i}"], p[f"W_d{i}"], p[f"W_t{i}"], p[f"W_id{i}"]])
           for i in range(2)]
    bvec = [jnp.stack([p[f"b_s{i}"], p[f"b_d{i}"], p[f"b_t{i}"], p[f"b_id{i}"],
                       p[f"bn_g{i}"], p[f"bn_b{i}"],
                       jnp.zeros((128,), f32), jnp.zeros((128,), f32)])
            for i in range(2)]
    w2p = jnp.zeros((128, 128), f32).at[:2].set(p["W_fc2"])
    fcb = jnp.zeros((8, 128), f32)
    fcb = fcb.at[0].set(p["b_fc1"])
    fcb = fcb.at[1, :2].set(p["b_fc2"])

    z64 = jnp.zeros((N_ACC, 64), f32)
    csrc_f, cdst_f, cnts = _sc_compact_kernel()(src_f, dst3_f)
    csrc = csrc_f.reshape(3, EBLK, 128)
    cdst = cdst_f.reshape(3, EBLK, 128)
    degparts = _sc_deg_kernel()(dst_r, wrow_r)
    tbl1, dis8 = _tc_prep(x_pad, degparts)
    mp1 = _sc_msg_kernel()(tbl1, csrc, cdst, cnts, z64)
    a1, st1 = _tc_combine(mp1, dis8, x_pad, wst[0], bvec[0])
    h1, tbl2 = _tc_norm(a1, st1, bvec[0], dis8)
    mp2 = _sc_msg_kernel()(tbl2, csrc, cdst, cnts, z64)
    a2, st2 = _tc_combine(mp2, dis8, h1, wst[1], bvec[1])
    _, res = _tc_final(a2, st2, bvec[1], batch_r, p["W_fc1"], w2p, fcb)
    return res[:, :2]
